# baseline (speedup 1.0000x reference)
import functools

import jax
import jax.numpy as jnp
from jax import lax
from jax.experimental import pallas as pl
from jax.experimental.pallas import tpu as pltpu
from jax.experimental.pallas import tpu_sc as plsc

N = 10000
E = 320000
D = 128
C = 16
ALPHA = 0.1
REG = 1e-05

NW = 32
NP = 320
N_PAD = NW * NP
CH = 128
NPT = N_PAD // 16
TRASH = 128
E_PAD = 327680
EB_A = 20
NB0 = 11
NB1 = 9
BK = 8 * CH
RAW_B = 304
TAIL0 = RAW_B * BK
RAW_T = (EB_A - 1) * BK


def _rsqrt16(v):
    i = lax.bitcast_convert_type(v, jnp.int32)
    i = jnp.int32(0x5F3759DF) - (i >> 1)
    y = lax.bitcast_convert_type(i, jnp.float32)
    for _ in range(3):
        y = y * (1.5 - 0.5 * v * y * y)
    return y


def _deg_dinv_s0_body(col_hbm, colt_hbm, x_hbm, dinv_hbm, s0_hbm,
                      deg_sh, zbuf, ones_v, ci_all, degv, xv, sem_i, sem_s):
    cid = lax.axis_index("c")
    sid = lax.axis_index("s")
    wid = cid * 16 + sid

    for g in range(NPT // 16):
        zbuf[pl.ds(16 * g, 16)] = jnp.zeros((16,), jnp.float32)
    pltpu.sync_copy(zbuf, deg_sh.at[pl.ds(sid * NPT, NPT)])
    pltpu.sync_copy(zbuf.at[pl.ds(0, 8)],
                    deg_sh.at[pl.ds(N_PAD + sid * 8, 8)])
    for g in range(CH // 16):
        ones_v[pl.ds(16 * g, 16)] = jnp.full((16,), 1.0, jnp.float32)
    plsc.subcore_barrier()

    d0 = pltpu.async_copy(col_hbm.at[pl.ds(sid * RAW_T, RAW_T)],
                          ci_all.at[pl.ds(0, RAW_T)], sem_i)
    d1 = pltpu.async_copy(colt_hbm.at[pl.ds(sid * BK, BK)],
                          ci_all.at[pl.ds(RAW_T, BK)], sem_i)
    d0.wait()
    d1.wait()
    descs = []
    for c in range(EB_A * 8):
        descs.append(pltpu.async_copy(
            ones_v, deg_sh.at[ci_all.at[pl.ds(c * CH, CH)]], sem_s, add=True))
    for dsc in descs:
        dsc.wait()
    plsc.subcore_barrier()

    base = jnp.minimum(wid * NP, N - NP)
    pltpu.sync_copy(deg_sh.at[pl.ds(base, NP)], degv)
    for g in range(NP // 16):
        dv = degv[pl.ds(16 * g, 16)] + 1.0
        degv[pl.ds(16 * g, 16)] = _rsqrt16(dv)
    pltpu.sync_copy(degv, dinv_hbm.at[pl.ds(base, NP)])

    pltpu.sync_copy(x_hbm.at[pl.ds(base, NP)], xv)

    def scale_group(g, _):
        dvec = degv[pl.ds(16 * g, 16)]
        for l in range(16):
            dv = dvec[l]
            r = 16 * g + l
            for j in range(D // 16):
                xv[r, pl.ds(16 * j, 16)] = xv[r, pl.ds(16 * j, 16)] * dv
        return _

    lax.fori_loop(0, NP // 16, scale_group, None)
    pltpu.sync_copy(xv, s0_hbm.at[pl.ds(base, NP)])


def _deg_dinv_s0(col_flat, col_tail, x):
    mesh = plsc.VectorSubcoreMesh(core_axis_name="c", subcore_axis_name="s")
    return pl.kernel(
        _deg_dinv_s0_body,
        out_type=(
            jax.ShapeDtypeStruct((N,), jnp.float32),
            jax.ShapeDtypeStruct((N, D), jnp.float32),
        ),
        mesh=mesh,
        scratch_types=[
            pltpu.VMEM_SHARED((N_PAD + TRASH,), jnp.float32),
            pltpu.VMEM((NPT,), jnp.float32),
            pltpu.VMEM((CH,), jnp.float32),
            pltpu.VMEM((EB_A * BK,), jnp.int32),
            pltpu.VMEM((NP,), jnp.float32),
            pltpu.VMEM((NP, D), jnp.float32),
            pltpu.SemaphoreType.DMA,
            pltpu.SemaphoreType.DMA,
        ],
    )(col_flat, col_tail, x)


def _hop_body(s_hbm, row_hbm, col_hbm, rowt_hbm, colt_hbm, tp_hbm,
              acc_sh, ri0, ci0, ri1, ci1, rows0, rows1,
              sem_i0, sem_i1, sem_g0, sem_g1):
    cid = lax.axis_index("c")
    sid = lax.axis_index("s")

    def zrow(r, _):
        for g in range(D // 16):
            rows0[r, pl.ds(16 * g, 16)] = jnp.zeros((16,), jnp.float32)
        return _

    lax.fori_loop(0, CH, zrow, None)
    for b in range(NPT // CH):
        pltpu.sync_copy(rows0, acc_sh.at[pl.ds(sid * NPT + b * CH, CH)])
    pltpu.sync_copy(rows0.at[pl.ds(0, 8)],
                    acc_sh.at[pl.ds(N_PAD + sid * 8, 8)])

    ibufs = ((ri0, ci0), (ri1, ci1))
    isems = (sem_i0, sem_i1)
    gbufs = (rows0, rows1)
    gsems = (sem_g0, sem_g1)

    def run_pipe(srcs):
        nblk = len(srcs)

        def fire_idx(blk, par):
            r_src, c_src = srcs[blk]()
            dr = pltpu.async_copy(r_src, ibufs[par][0], isems[par])
            dc = pltpu.async_copy(c_src, ibufs[par][1], isems[par])
            return (dr, dc)

        pend_i = [fire_idx(0, 0), None]
        for dsc in pend_i[0]:
            dsc.wait()
        if nblk > 1:
            pend_i[1] = fire_idx(1, 1)
        n_ch = nblk * 8
        pend_g = pltpu.async_copy(s_hbm.at[ri0.at[pl.ds(0, CH)]], rows0,
                                  sem_g0)
        for k in range(n_ch):
            b, t = divmod(k, 8)
            par = b % 2
            cur_buf = gbufs[k % 2]
            cur_dsc = pend_g
            if k + 1 < n_ch:
                nb, nt = divmod(k + 1, 8)
                if nt == 0:
                    for dsc in pend_i[nb % 2]:
                        dsc.wait()
                pend_g = pltpu.async_copy(
                    s_hbm.at[ibufs[nb % 2][0].at[pl.ds(nt * CH, CH)]],
                    gbufs[(k + 1) % 2], gsems[(k + 1) % 2])
            cur_dsc.wait()
            pltpu.sync_copy(
                cur_buf, acc_sh.at[ibufs[par][1].at[pl.ds(t * CH, CH)]],
                add=True)
            if t == 7 and b + 2 < nblk:
                pend_i[par] = fire_idx(b + 2, par)

    def raw(tb, blk):
        return lambda: (row_hbm.at[pl.ds((tb + blk) * BK, BK)],
                        col_hbm.at[pl.ds((tb + blk) * BK, BK)])

    def tail():
        return lambda: (rowt_hbm.at[pl.ds(sid * BK, BK)],
                        colt_hbm.at[pl.ds(sid * BK, BK)])

    @pl.when(cid == 0)
    def _():
        run_pipe([raw(sid * NB0, b) for b in range(NB0)])

    @pl.when(cid == 1)
    def _():
        run_pipe([raw(16 * NB0 + sid * (NB1 - 1), b)
                  for b in range(NB1 - 1)] + [tail()])

    plsc.subcore_barrier()

    pltpu.sync_copy(acc_sh.at[pl.ds(sid * NPT, NPT)],
                    tp_hbm.at[cid, pl.ds(sid * NPT, NPT)])


def _hop(s, row, col, row_tail, col_tail):
    mesh = plsc.VectorSubcoreMesh(core_axis_name="c", subcore_axis_name="s")
    return pl.kernel(
        _hop_body,
        out_type=jax.ShapeDtypeStruct((2, N_PAD, D), jnp.float32),
        mesh=mesh,
        scratch_types=[
            pltpu.VMEM_SHARED((N_PAD + TRASH, D), jnp.float32),
            pltpu.VMEM((BK,), jnp.int32),
            pltpu.VMEM((BK,), jnp.int32),
            pltpu.VMEM((BK,), jnp.int32),
            pltpu.VMEM((BK,), jnp.int32),
            pltpu.VMEM((CH, D), jnp.float32),
            pltpu.VMEM((CH, D), jnp.float32),
            pltpu.SemaphoreType.DMA,
            pltpu.SemaphoreType.DMA,
            pltpu.SemaphoreType.DMA,
            pltpu.SemaphoreType.DMA,
        ],
    )(s, row, col, row_tail, col_tail)


BN = 2000
NBLK = N // BN


def _combine1_body(tp, s, x, dinv, s_next):
    t = tp[0] + tp[1] + s[...]
    dv = dinv[...]
    out = (1.0 - ALPHA) * (dv * t) + ALPHA * x[...]
    s_next[...] = dv * out


def _combine1(tp, s, x, dinv_col):
    return pl.pallas_call(
        _combine1_body,
        grid=(NBLK,),
        in_specs=[
            pl.BlockSpec((2, BN, D), lambda i: (0, i, 0)),
            pl.BlockSpec((BN, D), lambda i: (i, 0)),
            pl.BlockSpec((BN, D), lambda i: (i, 0)),
            pl.BlockSpec((BN, 1), lambda i: (i, 0)),
        ],
        out_specs=pl.BlockSpec((BN, D), lambda i: (i, 0)),
        out_shape=jax.ShapeDtypeStruct((N, D), jnp.float32),
    )(tp, s, x, dinv_col)


def _combine_solve_body(tp, s, x, dinv, mask, yb, yp, xg_sc, acc_g, acc_r):
    i = pl.program_id(0)

    @pl.when(i == 0)
    def _():
        acc_g[...] = jnp.zeros_like(acc_g)
        acc_r[...] = jnp.zeros_like(acc_r)

    @pl.when(i < NBLK)
    def _():
        t = tp[0] + tp[1] + s[...]
        xg = (1.0 - ALPHA) * (dinv[...] * t) + ALPHA * x[...]
        xg_sc[pl.ds(i * BN, BN)] = xg
        xm = xg * mask[...]
        acc_g[...] += lax.dot_general(xm, xg, (((0,), (0,)), ((), ())),
                                      preferred_element_type=jnp.float32)
        acc_r[...] += lax.dot_general(xm, yb[...], (((0,), (0,)), ((), ())),
                                      preferred_element_type=jnp.float32)

    @pl.when(i == NBLK)
    def _():
        rows = lax.broadcasted_iota(jnp.int32, (D, D), 0)
        cols = lax.broadcasted_iota(jnp.int32, (D, D), 1)
        a = acc_g[...] + jnp.where(rows == cols, jnp.float32(REG), 0.0)
        aabs = jnp.abs(a)
        n1 = jnp.max(jnp.sum(aabs, axis=0))
        ninf = jnp.max(jnp.sum(aabs, axis=1))
        two_i = jnp.where(rows == cols, jnp.float32(2.0), jnp.float32(0.0))
        xinv = a * (1.0 / (n1 * ninf))
        for _ in range(24):
            ax = lax.dot_general(a, xinv, (((1,), (0,)), ((), ())),
                                 preferred_element_type=jnp.float32,
                                 precision=lax.Precision.HIGHEST)
            xinv = lax.dot_general(xinv, two_i - ax, (((1,), (0,)), ((), ())),
                                   preferred_element_type=jnp.float32,
                                   precision=lax.Precision.HIGHEST)
        sol = lax.dot_general(xinv, acc_r[...], (((1,), (0,)), ((), ())),
                              preferred_element_type=jnp.float32)
        yp[...] = lax.dot_general(xg_sc[...], sol, (((1,), (0,)), ((), ())),
                                  preferred_element_type=jnp.float32)


def _combine_solve(tp, s, x, dinv_col, mask_col, y_one_hot):
    clamp = lambda i: jnp.minimum(i, NBLK - 1)
    return pl.pallas_call(
        _combine_solve_body,
        grid=(NBLK + 1,),
        in_specs=[
            pl.BlockSpec((2, BN, D), lambda i: (0, clamp(i), 0)),
            pl.BlockSpec((BN, D), lambda i: (clamp(i), 0)),
            pl.BlockSpec((BN, D), lambda i: (clamp(i), 0)),
            pl.BlockSpec((BN, 1), lambda i: (clamp(i), 0)),
            pl.BlockSpec((BN, 1), lambda i: (clamp(i), 0)),
            pl.BlockSpec((BN, C), lambda i: (clamp(i), 0)),
        ],
        out_specs=pl.BlockSpec((N, C), lambda i: (0, 0)),
        out_shape=jax.ShapeDtypeStruct((N, C), jnp.float32),
        scratch_shapes=[
            pltpu.VMEM((N, D), jnp.float32),
            pltpu.VMEM((D, D), jnp.float32),
            pltpu.VMEM((D, C), jnp.float32),
        ],
    )(tp, s, x, dinv_col, mask_col, y_one_hot)


def kernel(x, edge_index, y_one_hot, train_mask):
    i = jnp.arange(16 * BK - (E - TAIL0), dtype=jnp.int32)
    row = edge_index[0]
    col = edge_index[1]
    row_tail = jnp.concatenate([row[TAIL0:], (i * 41) % N])
    col_tail = jnp.concatenate([col[TAIL0:], N_PAD + (i % TRASH)])
    mask_col = train_mask.astype(jnp.float32)[:, None]

    dinv, s0 = _deg_dinv_s0(col, col_tail, x)
    dinv_col = dinv[:, None]

    tp1 = _hop(s0, row, col, row_tail, col_tail)
    s1 = _combine1(tp1, s0, x, dinv_col)
    tp2 = _hop(s1, row, col, row_tail, col_tail)
    return _combine_solve(tp2, s1, x, dinv_col, mask_col, y_one_hot)
